# baseline (speedup 1.0000x reference)
import functools

import jax
import jax.numpy as jnp
from jax import lax
from jax.experimental import pallas as pl
from jax.experimental.pallas import tpu as pltpu
from jax.experimental.pallas import tpu_sc as plsc

B = 16384
K = 32
NC = 2
NS = 16
L = 16
NW = NC * NS
BPW = B // NW
CH = 128
NCH = BPW // CH


def _body(uid_hbm, iid_hbm, pi_hbm, eta_hbm, out_hbm,
          uid_v, iid_v, pi_v, eta_v, out_v, csum_v, gsem):
    wid = lax.axis_index("s") * NC + lax.axis_index("c")

    pltpu.sync_copy(uid_hbm.at[wid], uid_v)
    pltpu.sync_copy(iid_hbm.at[wid], iid_v)

    copies = []
    for j in range(NCH):
        copies.append(pltpu.async_copy(
            pi_hbm.at[uid_v.at[j]], pi_v.at[pl.ds(j * CH, CH)], gsem))
        copies.append(pltpu.async_copy(
            eta_hbm.at[iid_v.at[j]], eta_v.at[pl.ds(j * CH, CH)], gsem))
    for c in copies:
        c.wait()

    last_lane = lax.iota(jnp.int32, L) * L + (L - 1)

    def group(g, carry):
        base = g * L
        for j in range(L):
            r = base + j
            v = (pi_v[r, pl.ds(0, L)] * eta_v[r, pl.ds(0, L)]
                 + pi_v[r, pl.ds(L, L)] * eta_v[r, pl.ds(L, L)])
            csum_v[pl.ds(j * L, L)] = plsc.cumsum(v)
        sums = plsc.load_gather(csum_v, [last_lane])
        out_v[pl.ds(base, L)] = 1.0 - jnp.exp(-sums)
        return carry

    lax.fori_loop(0, BPW // L, group, 0)

    pltpu.sync_copy(out_v, out_hbm.at[pl.ds(wid * BPW, BPW)])


_pf = functools.partial(
    pl.kernel,
    mesh=plsc.VectorSubcoreMesh(core_axis_name="c", subcore_axis_name="s"),
    out_type=jax.ShapeDtypeStruct((B,), jnp.float32),
    compiler_params=pltpu.CompilerParams(
        needs_layout_passes=False, use_tc_tiling_on_sc=False),
    scratch_types=[
        pltpu.VMEM((NCH, CH), jnp.int32),
        pltpu.VMEM((NCH, CH), jnp.int32),
        pltpu.VMEM((BPW, K), jnp.float32),
        pltpu.VMEM((BPW, K), jnp.float32),
        pltpu.VMEM((BPW,), jnp.float32),
        pltpu.VMEM((L * L,), jnp.float32),
        pltpu.SemaphoreType.DMA,
    ],
)(_body)


def kernel(user_ids, item_ids, pi, eta):
    uid = user_ids.astype(jnp.int32).reshape(NW, NCH, CH)
    iid = item_ids.astype(jnp.int32).reshape(NW, NCH, CH)
    return _pf(uid, iid, pi, eta)


# speedup vs baseline: 2.2920x; 2.2920x over previous
import functools

import jax
import jax.numpy as jnp
from jax import lax
from jax.experimental import pallas as pl
from jax.experimental.pallas import tpu as pltpu
from jax.experimental.pallas import tpu_sc as plsc

B = 16384
K = 32
NC = 2
NS = 16
L = 16
NW = NC * NS
BPW = B // NW
CHK = 16
NCHK = BPW // CHK
NSLAB = 125000


def _body(uid_hbm, iid_hbm, pi3_hbm, eta3_hbm, out_hbm,
          uid_v, iid_v, out_v, stash_v,
          pi_a, pi_b, eta_a, eta_b, sem_a, sem_b):
    wid = lax.axis_index("s") * NC + lax.axis_index("c")

    pltpu.sync_copy(uid_hbm.at[wid], uid_v)
    pltpu.sync_copy(iid_hbm.at[wid], iid_v)

    last_lane = lax.iota(jnp.int32, L) * L + (L - 1)

    def fire(c, pi_buf, eta_buf, sem):
        uvec = uid_v[pl.ds(c * CHK, CHK)]
        tvec = iid_v[pl.ds(c * CHK, CHK)]
        for j in range(CHK):
            pltpu.async_copy(
                pi3_hbm.at[uvec[j] >> 3], pi_buf.at[j], sem)
            pltpu.async_copy(
                eta3_hbm.at[tvec[j] >> 3], eta_buf.at[j], sem)

    def drain(pi_buf, eta_buf, sem):
        pltpu.make_async_copy(pi3_hbm.at[pl.ds(0, CHK)], pi_buf, sem).wait()
        pltpu.make_async_copy(eta3_hbm.at[pl.ds(0, CHK)], eta_buf, sem).wait()

    def compute(c, pi_buf, eta_buf):
        uvec = uid_v[pl.ds(c * CHK, CHK)]
        tvec = iid_v[pl.ds(c * CHK, CHK)]
        for j in range(CHK):
            r = uvec[j] & 7
            s = tvec[j] & 7
            v = (pi_buf[j, r, pl.ds(0, L)] * eta_buf[j, s, pl.ds(0, L)]
                 + pi_buf[j, r, pl.ds(L, L)] * eta_buf[j, s, pl.ds(L, L)])
            stash_v[pl.ds(j * L, L)] = plsc.cumsum(v)
        sums = plsc.load_gather(stash_v, [last_lane])
        out_v[pl.ds(c * CHK, CHK)] = 1.0 - jnp.exp(-sums)

    fire(0, pi_a, eta_a, sem_a)

    def pair(p, carry):
        c0 = p * 2
        fire(c0 + 1, pi_b, eta_b, sem_b)
        drain(pi_a, eta_a, sem_a)
        compute(c0, pi_a, eta_a)

        @pl.when(p < NCHK // 2 - 1)
        def _():
            fire(c0 + 2, pi_a, eta_a, sem_a)

        drain(pi_b, eta_b, sem_b)
        compute(c0 + 1, pi_b, eta_b)
        return carry

    lax.fori_loop(0, NCHK // 2, pair, 0)

    pltpu.sync_copy(out_v, out_hbm.at[pl.ds(wid * BPW, BPW)])


_pf = functools.partial(
    pl.kernel,
    mesh=plsc.VectorSubcoreMesh(core_axis_name="c", subcore_axis_name="s"),
    out_type=jax.ShapeDtypeStruct((B,), jnp.float32),
    compiler_params=pltpu.CompilerParams(needs_layout_passes=False),
    scratch_types=[
        pltpu.VMEM((BPW,), jnp.int32),
        pltpu.VMEM((BPW,), jnp.int32),
        pltpu.VMEM((BPW,), jnp.float32),
        pltpu.VMEM((CHK * L,), jnp.float32),
        pltpu.VMEM((CHK, 8, K), jnp.float32),
        pltpu.VMEM((CHK, 8, K), jnp.float32),
        pltpu.VMEM((CHK, 8, K), jnp.float32),
        pltpu.VMEM((CHK, 8, K), jnp.float32),
        pltpu.SemaphoreType.DMA,
        pltpu.SemaphoreType.DMA,
    ],
)(_body)


def kernel(user_ids, item_ids, pi, eta):
    uid = user_ids.astype(jnp.int32).reshape(NW, BPW)
    iid = item_ids.astype(jnp.int32).reshape(NW, BPW)
    pi3 = pi.reshape(NSLAB, 8, K)
    eta3 = eta.reshape(NSLAB, 8, K)
    return _pf(uid, iid, pi3, eta3)
